# Initial kernel scaffold; baseline (speedup 1.0000x reference)
import jax
import jax.numpy as jnp
from jax.experimental import pallas as pl


def kernel(x, edge_index, W1, b1, W2, b2):
    raise NotImplementedError("write your pallas kernel here")


# speedup vs baseline: 28.5921x; 28.5921x over previous
import jax
import jax.numpy as jnp
from jax import lax
from jax.experimental import pallas as pl
from jax.experimental.pallas import tpu as pltpu
from jax.experimental.pallas import tpu_sc as plsc

N_NODES = 10000
N_EDGES = 320000
NC = 2
NS = 16
NW = NC * NS
CHUNK = 100
NCHUNKS_W = N_EDGES // NW // CHUNK
NCHUNKS_S = N_EDGES // NS // CHUNK
SLAB = 624
TAIL0 = NS * SLAB
TAIL = N_NODES - TAIL0
HIST_W = 16
HALF = 64


def _vector_mesh():
    return plsc.VectorSubcoreMesh(core_axis_name="c", subcore_axis_name="s")


_SC_PARAMS = pltpu.CompilerParams(use_tc_tiling_on_sc=False)


def _slab_copy(s, src_at, dst_at):
    row0 = pl.multiple_of(s * SLAB, 8)
    pltpu.sync_copy(src_at(pl.ds(row0, SLAB)), dst_at(pl.ds(row0, SLAB)))

    @pl.when(s == 0)
    def _():
        pltpu.sync_copy(src_at(pl.ds(TAIL0, TAIL)), dst_at(pl.ds(TAIL0, TAIL)))


def _hist_kernel(dst_hbm, ones_hbm, zeros_hbm, out_hbm, dst_v, ones_v, deg_sh):
    c = lax.axis_index("c")
    s = lax.axis_index("s")
    wid = s * NC + c
    pltpu.sync_copy(dst_hbm.at[wid], dst_v)
    pltpu.sync_copy(ones_hbm, ones_v)
    _slab_copy(s, lambda d: zeros_hbm.at[d], lambda d: deg_sh.at[d])
    plsc.subcore_barrier()

    @pl.loop(0, NCHUNKS_W)
    def _(j):
        pltpu.sync_copy(ones_v, deg_sh.at[dst_v.at[j]], add=True)

    plsc.subcore_barrier()
    _slab_copy(s, lambda d: deg_sh.at[d], lambda d: out_hbm.at[c, d])


def _degree_histogram(dst3w):
    ones = jnp.ones((CHUNK, HIST_W), jnp.float32)
    zeros = jnp.zeros((N_NODES, HIST_W), jnp.float32)
    k = pl.kernel(
        _hist_kernel,
        out_type=jax.ShapeDtypeStruct((NC, N_NODES, HIST_W), jnp.float32),
        mesh=_vector_mesh(),
        compiler_params=_SC_PARAMS,
        scratch_types=[
            pltpu.VMEM((NCHUNKS_W, CHUNK), jnp.int32),
            pltpu.VMEM((CHUNK, HIST_W), jnp.float32),
            pltpu.VMEM_SHARED((N_NODES, HIST_W), jnp.float32),
        ],
    )
    return k(dst3w, ones, zeros)


def _gather_scatter_loop(g_ref, src_v, dst_v, rows_v, acc_sh, sems, nchunks):
    for b in range(2):
        pltpu.async_copy(g_ref.at[src_v.at[b]], rows_v.at[b], sems.at[b])

    @pl.loop(0, nchunks, step=2)
    def _(j):
        for b in range(2):
            k = j + b
            pltpu.make_async_copy(g_ref.at[src_v.at[k]], rows_v.at[b],
                                  sems.at[b]).wait()
            pltpu.sync_copy(rows_v.at[b], acc_sh.at[dst_v.at[k]], add=True)

            @pl.when(k + 2 < nchunks)
            def _():
                pltpu.async_copy(g_ref.at[src_v.at[k + 2]], rows_v.at[b],
                                 sems.at[b])


def _conv1_kernel(g_hbm, src_hbm, dst_hbm, out_hbm, src_v, dst_v, rows_v,
                  acc_sh, sems):
    c = lax.axis_index("c")
    s = lax.axis_index("s")
    pltpu.sync_copy(src_hbm.at[s], src_v)
    pltpu.sync_copy(dst_hbm.at[s], dst_v)
    _slab_copy(s, lambda d: g_hbm.at[c, d], lambda d: acc_sh.at[d])
    plsc.subcore_barrier()
    _gather_scatter_loop(g_hbm.at[c], src_v, dst_v, rows_v, acc_sh, sems,
                         NCHUNKS_S)
    plsc.subcore_barrier()
    _slab_copy(s, lambda d: acc_sh.at[d], lambda d: out_hbm.at[c, d])


def _conv2_kernel(g_hbm, src_hbm, dst_hbm, out_hbm, src_v, dst_v, rows_v,
                  acc_sh, sems):
    c = lax.axis_index("c")
    s = lax.axis_index("s")
    wid = s * NC + c
    pltpu.sync_copy(src_hbm.at[wid], src_v)
    pltpu.sync_copy(dst_hbm.at[wid], dst_v)
    _slab_copy(s, lambda d: g_hbm.at[d], lambda d: acc_sh.at[d])
    plsc.subcore_barrier()
    _gather_scatter_loop(g_hbm, src_v, dst_v, rows_v, acc_sh, sems, NCHUNKS_W)
    plsc.subcore_barrier()
    _slab_copy(s, lambda d: acc_sh.at[d], lambda d: out_hbm.at[c, d])


def _aggregate1(g1h, src3s, dst3s):
    k = pl.kernel(
        _conv1_kernel,
        out_type=jax.ShapeDtypeStruct((NC, N_NODES, HALF), jnp.float32),
        mesh=_vector_mesh(),
        compiler_params=_SC_PARAMS,
        scratch_types=[
            pltpu.VMEM((NCHUNKS_S, CHUNK), jnp.int32),
            pltpu.VMEM((NCHUNKS_S, CHUNK), jnp.int32),
            pltpu.VMEM((2, CHUNK, HALF), jnp.float32),
            pltpu.VMEM_SHARED((N_NODES, HALF), jnp.float32),
            pltpu.SemaphoreType.DMA((2,)),
        ],
    )
    return k(g1h, src3s, dst3s)


def _aggregate2(g2, src3w, dst3w):
    width = g2.shape[1]
    k = pl.kernel(
        _conv2_kernel,
        out_type=jax.ShapeDtypeStruct((NC, N_NODES, width), jnp.float32),
        mesh=_vector_mesh(),
        compiler_params=_SC_PARAMS,
        scratch_types=[
            pltpu.VMEM((NCHUNKS_W, CHUNK), jnp.int32),
            pltpu.VMEM((NCHUNKS_W, CHUNK), jnp.int32),
            pltpu.VMEM((2, CHUNK, width), jnp.float32),
            pltpu.VMEM_SHARED((N_NODES, width), jnp.float32),
            pltpu.SemaphoreType.DMA((2,)),
        ],
    )
    return k(g2, src3w, dst3w)


_BLK = 1000


def _dinv_block(deg_ref):
    deg = deg_ref[0, :, 0:1] + deg_ref[1, :, 0:1] + 1.0
    return lax.rsqrt(deg)


def _scale_mm_kernel(x_ref, w_ref, deg_ref, o_ref):
    dinv = _dinv_block(deg_ref)
    g = dinv * jnp.dot(x_ref[...], w_ref[...],
                       preferred_element_type=jnp.float32)
    o_ref[0] = g[:, :HALF]
    o_ref[1] = g[:, HALF:]


def _mid_kernel(acc_ref, deg_ref, b_ref, w_ref, o_ref):
    dinv = _dinv_block(deg_ref)
    accfull = jnp.concatenate([acc_ref[0], acc_ref[1]], axis=1)
    z = dinv * accfull + b_ref[...]
    z = jnp.maximum(z, 0.0)
    o_ref[...] = dinv * jnp.dot(z, w_ref[...],
                                preferred_element_type=jnp.float32)


def _final_kernel(acc_ref, g_ref, deg_ref, b_ref, o_ref):
    dinv = _dinv_block(deg_ref)
    y = dinv * (acc_ref[0] + acc_ref[1] - g_ref[...]) + b_ref[...]
    m = jnp.max(y, axis=1, keepdims=True)
    e = jnp.exp(y - m)
    o_ref[...] = (y - m) - jnp.log(jnp.sum(e, axis=1, keepdims=True))


def _scaled_matmul1(x, w, degpair):
    din, dout = w.shape
    n = x.shape[0]
    return pl.pallas_call(
        _scale_mm_kernel,
        grid=(n // _BLK,),
        in_specs=[
            pl.BlockSpec((_BLK, din), lambda i: (i, 0)),
            pl.BlockSpec((din, dout), lambda i: (0, 0)),
            pl.BlockSpec((NC, _BLK, HIST_W), lambda i: (0, i, 0)),
        ],
        out_specs=pl.BlockSpec((NC, _BLK, HALF), lambda i: (0, i, 0)),
        out_shape=jax.ShapeDtypeStruct((NC, n, HALF), jnp.float32),
    )(x, w, degpair)


def _mid_stage(accpair, degpair, b, w):
    din, dout = w.shape
    n = accpair.shape[1]
    return pl.pallas_call(
        _mid_kernel,
        grid=(n // _BLK,),
        in_specs=[
            pl.BlockSpec((NC, _BLK, HALF), lambda i: (0, i, 0)),
            pl.BlockSpec((NC, _BLK, HIST_W), lambda i: (0, i, 0)),
            pl.BlockSpec((1, din), lambda i: (0, 0)),
            pl.BlockSpec((din, dout), lambda i: (0, 0)),
        ],
        out_specs=pl.BlockSpec((_BLK, dout), lambda i: (i, 0)),
        out_shape=jax.ShapeDtypeStruct((n, dout), jnp.float32),
    )(accpair, degpair, b.reshape(1, din), w)


def _final_stage(accpair, g, degpair, b):
    n, dout = g.shape
    return pl.pallas_call(
        _final_kernel,
        grid=(n // _BLK,),
        in_specs=[
            pl.BlockSpec((NC, _BLK, dout), lambda i: (0, i, 0)),
            pl.BlockSpec((_BLK, dout), lambda i: (i, 0)),
            pl.BlockSpec((NC, _BLK, HIST_W), lambda i: (0, i, 0)),
            pl.BlockSpec((1, dout), lambda i: (0, 0)),
        ],
        out_specs=pl.BlockSpec((_BLK, dout), lambda i: (i, 0)),
        out_shape=jax.ShapeDtypeStruct((n, dout), jnp.float32),
    )(accpair, g, degpair, b.reshape(1, dout))


@jax.jit
def kernel(x, edge_index, W1, b1, W2, b2):
    ei = edge_index.astype(jnp.int32)
    src3w = ei[0].reshape(NW, NCHUNKS_W, CHUNK)
    dst3w = ei[1].reshape(NW, NCHUNKS_W, CHUNK)
    src3s = ei[0].reshape(NS, NCHUNKS_S, CHUNK)
    dst3s = ei[1].reshape(NS, NCHUNKS_S, CHUNK)

    degpair = _degree_histogram(dst3w)
    g1h = _scaled_matmul1(x, W1, degpair)
    acc1 = _aggregate1(g1h, src3s, dst3s)
    g2 = _mid_stage(acc1, degpair, b1, W2)
    acc2 = _aggregate2(g2, src3w, dst3w)
    return _final_stage(acc2, g2, degpair, b2)


# speedup vs baseline: 32.6078x; 1.1404x over previous
import jax
import jax.numpy as jnp
from jax import lax
from jax.experimental import pallas as pl
from jax.experimental.pallas import tpu as pltpu
from jax.experimental.pallas import tpu_sc as plsc

N_NODES = 10000
N_EDGES = 320000
NC = 2
NS = 16
NW = NC * NS
CHUNK = 100
NCHUNKS_W = N_EDGES // NW // CHUNK

SLAB = 624
TAIL0 = NS * SLAB
TAIL = N_NODES - TAIL0
HIST_W = 16
HALF = 64


def _vector_mesh():
    return plsc.VectorSubcoreMesh(core_axis_name="c", subcore_axis_name="s")


_SC_PARAMS = pltpu.CompilerParams(use_tc_tiling_on_sc=False)


def _slab_copy(s, src_at, dst_at):
    row0 = pl.multiple_of(s * SLAB, 8)
    pltpu.sync_copy(src_at(pl.ds(row0, SLAB)), dst_at(pl.ds(row0, SLAB)))

    @pl.when(s == 0)
    def _():
        pltpu.sync_copy(src_at(pl.ds(TAIL0, TAIL)), dst_at(pl.ds(TAIL0, TAIL)))


def _hist_kernel(dst_hbm, ones_hbm, zeros_hbm, out_hbm, dst_v, ones_v, deg_sh):
    c = lax.axis_index("c")
    s = lax.axis_index("s")
    wid = s * NC + c
    pltpu.sync_copy(dst_hbm.at[wid], dst_v)
    pltpu.sync_copy(ones_hbm, ones_v)
    _slab_copy(s, lambda d: zeros_hbm.at[d], lambda d: deg_sh.at[d])
    plsc.subcore_barrier()

    @pl.loop(0, NCHUNKS_W)
    def _(j):
        pltpu.sync_copy(ones_v, deg_sh.at[dst_v.at[j]], add=True)

    plsc.subcore_barrier()
    _slab_copy(s, lambda d: deg_sh.at[d], lambda d: out_hbm.at[c, d])


def _degree_histogram(dst3w):
    ones = jnp.ones((CHUNK, HIST_W), jnp.float32)
    zeros = jnp.zeros((N_NODES, HIST_W), jnp.float32)
    k = pl.kernel(
        _hist_kernel,
        out_type=jax.ShapeDtypeStruct((NC, N_NODES, HIST_W), jnp.float32),
        mesh=_vector_mesh(),
        compiler_params=_SC_PARAMS,
        scratch_types=[
            pltpu.VMEM((NCHUNKS_W, CHUNK), jnp.int32),
            pltpu.VMEM((CHUNK, HIST_W), jnp.float32),
            pltpu.VMEM_SHARED((N_NODES, HIST_W), jnp.float32),
        ],
    )
    return k(dst3w, ones, zeros)


def _gather_scatter_loop(g_ref, src_v, dst_v, rows_v, acc_sh, sems, nchunks):
    for b in range(2):
        pltpu.async_copy(g_ref.at[src_v.at[b]], rows_v.at[b], sems.at[b])

    @pl.loop(0, nchunks, step=2)
    def _(j):
        for b in range(2):
            k = j + b
            pltpu.make_async_copy(g_ref.at[src_v.at[k]], rows_v.at[b],
                                  sems.at[b]).wait()
            pltpu.sync_copy(rows_v.at[b], acc_sh.at[dst_v.at[k]], add=True)

            @pl.when(k + 2 < nchunks)
            def _():
                pltpu.async_copy(g_ref.at[src_v.at[k + 2]], rows_v.at[b],
                                 sems.at[b])


def _make_conv_kernel(nchunks):
    def _conv_kernel(g_hbm, src_hbm, dst_hbm, out_hbm, src_v, dst_v, rows_v,
                     acc_sh, sems):
        c = lax.axis_index("c")
        s = lax.axis_index("s")
        wid = s * NC + c
        pltpu.sync_copy(src_hbm.at[wid], src_v)
        pltpu.sync_copy(dst_hbm.at[wid], dst_v)
        _slab_copy(s, lambda d: g_hbm.at[d], lambda d: acc_sh.at[d])
        plsc.subcore_barrier()
        _gather_scatter_loop(g_hbm, src_v, dst_v, rows_v, acc_sh, sems,
                             nchunks)
        plsc.subcore_barrier()
        _slab_copy(s, lambda d: acc_sh.at[d], lambda d: out_hbm.at[c, d])

    return _conv_kernel


def _aggregate(g, src3, dst3):
    width = g.shape[1]
    nchunks, chunk = src3.shape[1], src3.shape[2]
    k = pl.kernel(
        _make_conv_kernel(nchunks),
        out_type=jax.ShapeDtypeStruct((NC, N_NODES, width), jnp.float32),
        mesh=_vector_mesh(),
        compiler_params=_SC_PARAMS,
        scratch_types=[
            pltpu.VMEM((nchunks, chunk), jnp.int32),
            pltpu.VMEM((nchunks, chunk), jnp.int32),
            pltpu.VMEM((2, chunk, width), jnp.float32),
            pltpu.VMEM_SHARED((N_NODES, width), jnp.float32),
            pltpu.SemaphoreType.DMA((2,)),
        ],
    )
    return k(g, src3, dst3)


_BLK = 1000


def _dinv_block(deg_ref):
    deg = deg_ref[0, :, 0:1] + deg_ref[1, :, 0:1] + 1.0
    return lax.rsqrt(deg)


def _scale_mm_kernel(x_ref, w_ref, deg_ref, o_ref):
    dinv = _dinv_block(deg_ref)
    o_ref[...] = dinv * jnp.dot(x_ref[...], w_ref[...],
                                preferred_element_type=jnp.float32)


def _mid_kernel(acc_ref, g_ref, deg_ref, b_ref, w_ref, o_ref):
    dinv = _dinv_block(deg_ref)
    z = dinv * (acc_ref[0] + acc_ref[1] - g_ref[...]) + b_ref[...]
    z = jnp.maximum(z, 0.0)
    o_ref[...] = dinv * jnp.dot(z, w_ref[...],
                                preferred_element_type=jnp.float32)


def _final_kernel(acc_ref, g_ref, deg_ref, b_ref, o_ref):
    dinv = _dinv_block(deg_ref)
    y = dinv * (acc_ref[0] + acc_ref[1] - g_ref[...]) + b_ref[...]
    m = jnp.max(y, axis=1, keepdims=True)
    e = jnp.exp(y - m)
    o_ref[...] = (y - m) - jnp.log(jnp.sum(e, axis=1, keepdims=True))


def _scaled_matmul1(x, w, degpair):
    din, dout = w.shape
    n = x.shape[0]
    return pl.pallas_call(
        _scale_mm_kernel,
        grid=(n // _BLK,),
        in_specs=[
            pl.BlockSpec((_BLK, din), lambda i: (i, 0)),
            pl.BlockSpec((din, dout), lambda i: (0, 0)),
            pl.BlockSpec((NC, _BLK, HIST_W), lambda i: (0, i, 0)),
        ],
        out_specs=pl.BlockSpec((_BLK, dout), lambda i: (i, 0)),
        out_shape=jax.ShapeDtypeStruct((n, dout), jnp.float32),
    )(x, w, degpair)


def _mid_stage(accpair, g, degpair, b, w):
    din, dout = w.shape
    n = accpair.shape[1]
    return pl.pallas_call(
        _mid_kernel,
        grid=(n // _BLK,),
        in_specs=[
            pl.BlockSpec((NC, _BLK, din), lambda i: (0, i, 0)),
            pl.BlockSpec((_BLK, din), lambda i: (i, 0)),
            pl.BlockSpec((NC, _BLK, HIST_W), lambda i: (0, i, 0)),
            pl.BlockSpec((1, din), lambda i: (0, 0)),
            pl.BlockSpec((din, dout), lambda i: (0, 0)),
        ],
        out_specs=pl.BlockSpec((_BLK, dout), lambda i: (i, 0)),
        out_shape=jax.ShapeDtypeStruct((n, dout), jnp.float32),
    )(accpair, g, degpair, b.reshape(1, din), w)


def _final_stage(accpair, g, degpair, b):
    n, dout = g.shape
    return pl.pallas_call(
        _final_kernel,
        grid=(n // _BLK,),
        in_specs=[
            pl.BlockSpec((NC, _BLK, dout), lambda i: (0, i, 0)),
            pl.BlockSpec((_BLK, dout), lambda i: (i, 0)),
            pl.BlockSpec((NC, _BLK, HIST_W), lambda i: (0, i, 0)),
            pl.BlockSpec((1, dout), lambda i: (0, 0)),
        ],
        out_specs=pl.BlockSpec((_BLK, dout), lambda i: (i, 0)),
        out_shape=jax.ShapeDtypeStruct((n, dout), jnp.float32),
    )(accpair, g, degpair, b.reshape(1, dout))


@jax.jit
def kernel(x, edge_index, W1, b1, W2, b2):
    ei = edge_index.astype(jnp.int32)
    src3w = ei[0].reshape(NW, NCHUNKS_W, CHUNK)
    dst3w = ei[1].reshape(NW, NCHUNKS_W, CHUNK)
    degpair = _degree_histogram(dst3w)
    g1 = _scaled_matmul1(x, W1, degpair)
    acc1 = _aggregate(g1, src3w, dst3w)
    g2 = _mid_stage(acc1, g1, degpair, b1, W2)
    acc2 = _aggregate(g2, src3w, dst3w)
    return _final_stage(acc2, g2, degpair, b2)


# speedup vs baseline: 33.4429x; 1.0256x over previous
import jax
import jax.numpy as jnp
from jax import lax
from jax.experimental import pallas as pl
from jax.experimental.pallas import tpu as pltpu
from jax.experimental.pallas import tpu_sc as plsc

N_NODES = 10000
N_EDGES = 320000
NC = 2
NS = 16
NW = NC * NS
CHUNK = 100
NCHUNKS_W = N_EDGES // NW // CHUNK
CHUNK2 = 125
NCHUNKS2_W = N_EDGES // NW // CHUNK2

SLAB = 624
TAIL0 = NS * SLAB
TAIL = N_NODES - TAIL0
HIST_W = 16
HALF = 64


def _vector_mesh():
    return plsc.VectorSubcoreMesh(core_axis_name="c", subcore_axis_name="s")


_SC_PARAMS = pltpu.CompilerParams(use_tc_tiling_on_sc=False)


def _slab_copy(s, src_at, dst_at):
    row0 = pl.multiple_of(s * SLAB, 8)
    pltpu.sync_copy(src_at(pl.ds(row0, SLAB)), dst_at(pl.ds(row0, SLAB)))

    @pl.when(s == 0)
    def _():
        pltpu.sync_copy(src_at(pl.ds(TAIL0, TAIL)), dst_at(pl.ds(TAIL0, TAIL)))


def _hist_kernel(dst_hbm, ones_hbm, zeros_hbm, out_hbm, dst_v, ones_v, deg_sh):
    c = lax.axis_index("c")
    s = lax.axis_index("s")
    wid = s * NC + c
    pltpu.sync_copy(dst_hbm.at[wid], dst_v)
    pltpu.sync_copy(ones_hbm, ones_v)
    _slab_copy(s, lambda d: zeros_hbm.at[d], lambda d: deg_sh.at[d])
    plsc.subcore_barrier()

    @pl.loop(0, NCHUNKS2_W)
    def _(j):
        pltpu.sync_copy(ones_v, deg_sh.at[dst_v.at[j]], add=True)

    plsc.subcore_barrier()
    _slab_copy(s, lambda d: deg_sh.at[d], lambda d: out_hbm.at[c, d])


def _degree_histogram(dst3w):
    ones = jnp.ones((CHUNK2, HIST_W), jnp.float32)
    zeros = jnp.zeros((N_NODES, HIST_W), jnp.float32)
    k = pl.kernel(
        _hist_kernel,
        out_type=jax.ShapeDtypeStruct((NC, N_NODES, HIST_W), jnp.float32),
        mesh=_vector_mesh(),
        compiler_params=_SC_PARAMS,
        scratch_types=[
            pltpu.VMEM((NCHUNKS2_W, CHUNK2), jnp.int32),
            pltpu.VMEM((CHUNK2, HIST_W), jnp.float32),
            pltpu.VMEM_SHARED((N_NODES, HIST_W), jnp.float32),
        ],
    )
    return k(dst3w, ones, zeros)


def _gather_scatter_loop(g_ref, src_v, dst_v, rows_v, acc_sh, sems, nchunks):
    for b in range(2):
        pltpu.async_copy(g_ref.at[src_v.at[b]], rows_v.at[b], sems.at[b])

    @pl.loop(0, nchunks, step=2)
    def _(j):
        for b in range(2):
            k = j + b
            pltpu.make_async_copy(g_ref.at[src_v.at[k]], rows_v.at[b],
                                  sems.at[b]).wait()
            pltpu.sync_copy(rows_v.at[b], acc_sh.at[dst_v.at[k]], add=True)

            @pl.when(k + 2 < nchunks)
            def _():
                pltpu.async_copy(g_ref.at[src_v.at[k + 2]], rows_v.at[b],
                                 sems.at[b])


def _make_conv_kernel(nchunks):
    def _conv_kernel(g_hbm, src_hbm, dst_hbm, out_hbm, src_v, dst_v, rows_v,
                     acc_sh, sems):
        c = lax.axis_index("c")
        s = lax.axis_index("s")
        wid = s * NC + c
        pltpu.sync_copy(src_hbm.at[wid], src_v)
        pltpu.sync_copy(dst_hbm.at[wid], dst_v)
        _slab_copy(s, lambda d: g_hbm.at[d], lambda d: acc_sh.at[d])
        plsc.subcore_barrier()
        _gather_scatter_loop(g_hbm, src_v, dst_v, rows_v, acc_sh, sems,
                             nchunks)
        plsc.subcore_barrier()
        _slab_copy(s, lambda d: acc_sh.at[d], lambda d: out_hbm.at[c, d])

    return _conv_kernel


def _aggregate(g, src3, dst3):
    width = g.shape[1]
    nchunks, chunk = src3.shape[1], src3.shape[2]
    k = pl.kernel(
        _make_conv_kernel(nchunks),
        out_type=jax.ShapeDtypeStruct((NC, N_NODES, width), jnp.float32),
        mesh=_vector_mesh(),
        compiler_params=_SC_PARAMS,
        scratch_types=[
            pltpu.VMEM((nchunks, chunk), jnp.int32),
            pltpu.VMEM((nchunks, chunk), jnp.int32),
            pltpu.VMEM((2, chunk, width), jnp.float32),
            pltpu.VMEM_SHARED((N_NODES, width), jnp.float32),
            pltpu.SemaphoreType.DMA((2,)),
        ],
    )
    return k(g, src3, dst3)


_BLK = 1000


def _dinv_block(deg_ref):
    deg = deg_ref[0, :, 0:1] + deg_ref[1, :, 0:1] + 1.0
    return lax.rsqrt(deg)


def _scale_mm_kernel(x_ref, w_ref, deg_ref, o_ref):
    dinv = _dinv_block(deg_ref)
    o_ref[...] = dinv * jnp.dot(x_ref[...], w_ref[...],
                                preferred_element_type=jnp.float32)


def _mid_kernel(acc_ref, g_ref, deg_ref, b_ref, w_ref, o_ref):
    dinv = _dinv_block(deg_ref)
    z = dinv * (acc_ref[0] + acc_ref[1] - g_ref[...]) + b_ref[...]
    z = jnp.maximum(z, 0.0)
    o_ref[...] = dinv * jnp.dot(z, w_ref[...],
                                preferred_element_type=jnp.float32)


def _final_kernel(acc_ref, g_ref, deg_ref, b_ref, o_ref):
    dinv = _dinv_block(deg_ref)
    y = dinv * (acc_ref[0] + acc_ref[1] - g_ref[...]) + b_ref[...]
    m = jnp.max(y, axis=1, keepdims=True)
    e = jnp.exp(y - m)
    o_ref[...] = (y - m) - jnp.log(jnp.sum(e, axis=1, keepdims=True))


def _scaled_matmul1(x, w, degpair):
    din, dout = w.shape
    n = x.shape[0]
    return pl.pallas_call(
        _scale_mm_kernel,
        grid=(n // _BLK,),
        in_specs=[
            pl.BlockSpec((_BLK, din), lambda i: (i, 0)),
            pl.BlockSpec((din, dout), lambda i: (0, 0)),
            pl.BlockSpec((NC, _BLK, HIST_W), lambda i: (0, i, 0)),
        ],
        out_specs=pl.BlockSpec((_BLK, dout), lambda i: (i, 0)),
        out_shape=jax.ShapeDtypeStruct((n, dout), jnp.float32),
    )(x, w, degpair)


def _mid_stage(accpair, g, degpair, b, w):
    din, dout = w.shape
    n = accpair.shape[1]
    return pl.pallas_call(
        _mid_kernel,
        grid=(n // _BLK,),
        in_specs=[
            pl.BlockSpec((NC, _BLK, din), lambda i: (0, i, 0)),
            pl.BlockSpec((_BLK, din), lambda i: (i, 0)),
            pl.BlockSpec((NC, _BLK, HIST_W), lambda i: (0, i, 0)),
            pl.BlockSpec((1, din), lambda i: (0, 0)),
            pl.BlockSpec((din, dout), lambda i: (0, 0)),
        ],
        out_specs=pl.BlockSpec((_BLK, dout), lambda i: (i, 0)),
        out_shape=jax.ShapeDtypeStruct((n, dout), jnp.float32),
    )(accpair, g, degpair, b.reshape(1, din), w)


def _final_stage(accpair, g, degpair, b):
    n, dout = g.shape
    return pl.pallas_call(
        _final_kernel,
        grid=(n // _BLK,),
        in_specs=[
            pl.BlockSpec((NC, _BLK, dout), lambda i: (0, i, 0)),
            pl.BlockSpec((_BLK, dout), lambda i: (i, 0)),
            pl.BlockSpec((NC, _BLK, HIST_W), lambda i: (0, i, 0)),
            pl.BlockSpec((1, dout), lambda i: (0, 0)),
        ],
        out_specs=pl.BlockSpec((_BLK, dout), lambda i: (i, 0)),
        out_shape=jax.ShapeDtypeStruct((n, dout), jnp.float32),
    )(accpair, g, degpair, b.reshape(1, dout))


@jax.jit
def kernel(x, edge_index, W1, b1, W2, b2):
    ei = edge_index.astype(jnp.int32)
    src3w = ei[0].reshape(NW, NCHUNKS_W, CHUNK)
    dst3w = ei[1].reshape(NW, NCHUNKS_W, CHUNK)
    src3b = ei[0].reshape(NW, NCHUNKS2_W, CHUNK2)
    dst3b = ei[1].reshape(NW, NCHUNKS2_W, CHUNK2)
    degpair = _degree_histogram(dst3b)
    g1 = _scaled_matmul1(x, W1, degpair)
    acc1 = _aggregate(g1, src3w, dst3w)
    g2 = _mid_stage(acc1, g1, degpair, b1, W2)
    acc2 = _aggregate(g2, src3b, dst3b)
    return _final_stage(acc2, g2, degpair, b2)


# speedup vs baseline: 33.5149x; 1.0022x over previous
import jax
import jax.numpy as jnp
from jax import lax
from jax.experimental import pallas as pl
from jax.experimental.pallas import tpu as pltpu
from jax.experimental.pallas import tpu_sc as plsc

N_NODES = 10000
N_EDGES = 320000
NC = 2
NS = 16
NW = NC * NS
CHUNK = 100
NCHUNKS_W = N_EDGES // NW // CHUNK
CHUNK2 = 125
NCHUNKS2_W = N_EDGES // NW // CHUNK2

SLAB = 624
TAIL0 = NS * SLAB
TAIL = N_NODES - TAIL0
HIST_W = 16
HALF = 64


def _vector_mesh():
    return plsc.VectorSubcoreMesh(core_axis_name="c", subcore_axis_name="s")


_SC_PARAMS = pltpu.CompilerParams(use_tc_tiling_on_sc=False)


def _slab_copy(s, src_at, dst_at):
    row0 = pl.multiple_of(s * SLAB, 8)
    pltpu.sync_copy(src_at(pl.ds(row0, SLAB)), dst_at(pl.ds(row0, SLAB)))

    @pl.when(s == 0)
    def _():
        pltpu.sync_copy(src_at(pl.ds(TAIL0, TAIL)), dst_at(pl.ds(TAIL0, TAIL)))


def _hist_kernel(dst_hbm, ones_hbm, zeros_hbm, out_hbm, dst_v, ones_v, deg_sh):
    c = lax.axis_index("c")
    s = lax.axis_index("s")
    wid = s * NC + c
    pltpu.sync_copy(dst_hbm.at[wid], dst_v)
    pltpu.sync_copy(ones_hbm, ones_v)
    _slab_copy(s, lambda d: zeros_hbm.at[d], lambda d: deg_sh.at[d])
    plsc.subcore_barrier()

    @pl.loop(0, NCHUNKS2_W)
    def _(j):
        pltpu.sync_copy(ones_v, deg_sh.at[dst_v.at[j]], add=True)

    plsc.subcore_barrier()
    _slab_copy(s, lambda d: deg_sh.at[d], lambda d: out_hbm.at[c, d])


def _degree_histogram(dst3w):
    ones = jnp.ones((CHUNK2, HIST_W), jnp.float32)
    zeros = jnp.zeros((N_NODES, HIST_W), jnp.float32)
    k = pl.kernel(
        _hist_kernel,
        out_type=jax.ShapeDtypeStruct((NC, N_NODES, HIST_W), jnp.float32),
        mesh=_vector_mesh(),
        compiler_params=_SC_PARAMS,
        scratch_types=[
            pltpu.VMEM((NCHUNKS2_W, CHUNK2), jnp.int32),
            pltpu.VMEM((CHUNK2, HIST_W), jnp.float32),
            pltpu.VMEM_SHARED((N_NODES, HIST_W), jnp.float32),
        ],
    )
    return k(dst3w, ones, zeros)


def _gather_scatter_loop(g_ref, src_v, dst_v, rows_v, acc_sh, sems, nchunks):
    for b in range(2):
        pltpu.async_copy(g_ref.at[src_v.at[b]], rows_v.at[b], sems.at[b])

    @pl.loop(0, nchunks, step=2)
    def _(j):
        for b in range(2):
            k = j + b
            pltpu.make_async_copy(g_ref.at[src_v.at[k]], rows_v.at[b],
                                  sems.at[b]).wait()
            pltpu.sync_copy(rows_v.at[b], acc_sh.at[dst_v.at[k]], add=True)

            @pl.when(k + 2 < nchunks)
            def _():
                pltpu.async_copy(g_ref.at[src_v.at[k + 2]], rows_v.at[b],
                                 sems.at[b])


def _make_conv_kernel(nchunks):
    def _conv_kernel(g_hbm, src_hbm, dst_hbm, out_hbm, src_v, dst_v, rows_v,
                     acc_sh, sems):
        c = lax.axis_index("c")
        s = lax.axis_index("s")
        wid = s * NC + c
        pltpu.sync_copy(src_hbm.at[wid], src_v)
        pltpu.sync_copy(dst_hbm.at[wid], dst_v)
        _slab_copy(s, lambda d: g_hbm.at[d], lambda d: acc_sh.at[d])
        plsc.subcore_barrier()
        _gather_scatter_loop(g_hbm, src_v, dst_v, rows_v, acc_sh, sems,
                             nchunks)
        plsc.subcore_barrier()
        _slab_copy(s, lambda d: acc_sh.at[d], lambda d: out_hbm.at[c, d])

    return _conv_kernel


def _aggregate(g, src3, dst3):
    width = g.shape[1]
    nchunks, chunk = src3.shape[1], src3.shape[2]
    k = pl.kernel(
        _make_conv_kernel(nchunks),
        out_type=jax.ShapeDtypeStruct((NC, N_NODES, width), jnp.float32),
        mesh=_vector_mesh(),
        compiler_params=_SC_PARAMS,
        scratch_types=[
            pltpu.VMEM((nchunks, chunk), jnp.int32),
            pltpu.VMEM((nchunks, chunk), jnp.int32),
            pltpu.VMEM((2, chunk, width), jnp.float32),
            pltpu.VMEM_SHARED((N_NODES, width), jnp.float32),
            pltpu.SemaphoreType.DMA((2,)),
        ],
    )
    return k(g, src3, dst3)


_BLK = 1000


def _dinv_block(deg_ref):
    deg = deg_ref[0, :, 0:1] + deg_ref[1, :, 0:1] + 1.0
    return lax.rsqrt(deg)


def _mm_kernel(x_ref, w_ref, o_ref):
    o_ref[...] = jnp.dot(x_ref[...], w_ref[...],
                         preferred_element_type=jnp.float32)


def _scale_kernel(h_ref, deg_ref, o_ref):
    o_ref[...] = _dinv_block(deg_ref) * h_ref[...]


def _mid_kernel(acc_ref, g_ref, deg_ref, b_ref, w_ref, o_ref):
    dinv = _dinv_block(deg_ref)
    z = dinv * (acc_ref[0] + acc_ref[1] - g_ref[...]) + b_ref[...]
    z = jnp.maximum(z, 0.0)
    o_ref[...] = dinv * jnp.dot(z, w_ref[...],
                                preferred_element_type=jnp.float32)


def _final_kernel(acc_ref, g_ref, deg_ref, b_ref, o_ref):
    dinv = _dinv_block(deg_ref)
    y = dinv * (acc_ref[0] + acc_ref[1] - g_ref[...]) + b_ref[...]
    m = jnp.max(y, axis=1, keepdims=True)
    e = jnp.exp(y - m)
    o_ref[...] = (y - m) - jnp.log(jnp.sum(e, axis=1, keepdims=True))


def _matmul1(x, w):
    din, dout = w.shape
    n = x.shape[0]
    return pl.pallas_call(
        _mm_kernel,
        grid=(n // _BLK,),
        in_specs=[
            pl.BlockSpec((_BLK, din), lambda i: (i, 0)),
            pl.BlockSpec((din, dout), lambda i: (0, 0)),
        ],
        out_specs=pl.BlockSpec((_BLK, dout), lambda i: (i, 0)),
        out_shape=jax.ShapeDtypeStruct((n, dout), jnp.float32),
    )(x, w)


def _scale_stage(h, degpair):
    n, dout = h.shape
    return pl.pallas_call(
        _scale_kernel,
        grid=(n // _BLK,),
        in_specs=[
            pl.BlockSpec((_BLK, dout), lambda i: (i, 0)),
            pl.BlockSpec((NC, _BLK, HIST_W), lambda i: (0, i, 0)),
        ],
        out_specs=pl.BlockSpec((_BLK, dout), lambda i: (i, 0)),
        out_shape=jax.ShapeDtypeStruct((n, dout), jnp.float32),
    )(h, degpair)


def _mid_stage(accpair, g, degpair, b, w):
    din, dout = w.shape
    n = accpair.shape[1]
    return pl.pallas_call(
        _mid_kernel,
        grid=(n // _BLK,),
        in_specs=[
            pl.BlockSpec((NC, _BLK, din), lambda i: (0, i, 0)),
            pl.BlockSpec((_BLK, din), lambda i: (i, 0)),
            pl.BlockSpec((NC, _BLK, HIST_W), lambda i: (0, i, 0)),
            pl.BlockSpec((1, din), lambda i: (0, 0)),
            pl.BlockSpec((din, dout), lambda i: (0, 0)),
        ],
        out_specs=pl.BlockSpec((_BLK, dout), lambda i: (i, 0)),
        out_shape=jax.ShapeDtypeStruct((n, dout), jnp.float32),
    )(accpair, g, degpair, b.reshape(1, din), w)


def _final_stage(accpair, g, degpair, b):
    n, dout = g.shape
    return pl.pallas_call(
        _final_kernel,
        grid=(n // _BLK,),
        in_specs=[
            pl.BlockSpec((NC, _BLK, dout), lambda i: (0, i, 0)),
            pl.BlockSpec((_BLK, dout), lambda i: (i, 0)),
            pl.BlockSpec((NC, _BLK, HIST_W), lambda i: (0, i, 0)),
            pl.BlockSpec((1, dout), lambda i: (0, 0)),
        ],
        out_specs=pl.BlockSpec((_BLK, dout), lambda i: (i, 0)),
        out_shape=jax.ShapeDtypeStruct((n, dout), jnp.float32),
    )(accpair, g, degpair, b.reshape(1, dout))


@jax.jit
def kernel(x, edge_index, W1, b1, W2, b2):
    ei = edge_index.astype(jnp.int32)
    src3w = ei[0].reshape(NW, NCHUNKS_W, CHUNK)
    dst3w = ei[1].reshape(NW, NCHUNKS_W, CHUNK)
    src3b = ei[0].reshape(NW, NCHUNKS2_W, CHUNK2)
    dst3b = ei[1].reshape(NW, NCHUNKS2_W, CHUNK2)
    h = _matmul1(x, W1)
    degpair = _degree_histogram(dst3b)
    g1 = _scale_stage(h, degpair)
    acc1 = _aggregate(g1, src3w, dst3w)
    g2 = _mid_stage(acc1, g1, degpair, b1, W2)
    acc2 = _aggregate(g2, src3b, dst3b)
    return _final_stage(acc2, g2, degpair, b2)


# speedup vs baseline: 36.8167x; 1.0985x over previous
import jax
import jax.numpy as jnp
from jax import lax
from jax.experimental import pallas as pl
from jax.experimental.pallas import tpu as pltpu
from jax.experimental.pallas import tpu_sc as plsc

N_NODES = 10000
N_EDGES = 320000
NC = 2
NS = 16
NW = NC * NS
CHUNK2 = 125
NCHUNKS2_W = N_EDGES // NW // CHUNK2

SLAB = 624
TAIL0 = NS * SLAB
TAIL = N_NODES - TAIL0
HIST_W = 16
HALF = 64


def _vector_mesh():
    return plsc.VectorSubcoreMesh(core_axis_name="c", subcore_axis_name="s")


_SC_PARAMS = pltpu.CompilerParams(use_tc_tiling_on_sc=False)


def _slab_copy(s, src_at, dst_at):
    row0 = pl.multiple_of(s * SLAB, 8)
    pltpu.sync_copy(src_at(pl.ds(row0, SLAB)), dst_at(pl.ds(row0, SLAB)))

    @pl.when(s == 0)
    def _():
        pltpu.sync_copy(src_at(pl.ds(TAIL0, TAIL)), dst_at(pl.ds(TAIL0, TAIL)))


def _hist_kernel(dst_hbm, ones_hbm, zeros_hbm, out_hbm, dst_v, ones_v, deg_sh):
    c = lax.axis_index("c")
    s = lax.axis_index("s")
    wid = s * NC + c
    pltpu.sync_copy(dst_hbm.at[wid], dst_v)
    pltpu.sync_copy(ones_hbm, ones_v)
    _slab_copy(s, lambda d: zeros_hbm.at[d], lambda d: deg_sh.at[d])
    plsc.subcore_barrier()

    @pl.loop(0, NCHUNKS2_W)
    def _(j):
        pltpu.sync_copy(ones_v, deg_sh.at[dst_v.at[j]], add=True)

    plsc.subcore_barrier()
    _slab_copy(s, lambda d: deg_sh.at[d], lambda d: out_hbm.at[c, d])


def _degree_histogram(dst3w):
    ones = jnp.ones((CHUNK2, HIST_W), jnp.float32)
    zeros = jnp.zeros((N_NODES, HIST_W), jnp.float32)
    k = pl.kernel(
        _hist_kernel,
        out_type=jax.ShapeDtypeStruct((NC, N_NODES, HIST_W), jnp.float32),
        mesh=_vector_mesh(),
        compiler_params=_SC_PARAMS,
        scratch_types=[
            pltpu.VMEM((NCHUNKS2_W, CHUNK2), jnp.int32),
            pltpu.VMEM((CHUNK2, HIST_W), jnp.float32),
            pltpu.VMEM_SHARED((N_NODES, HIST_W), jnp.float32),
        ],
    )
    return k(dst3w, ones, zeros)


def _gather_scatter_loop(g_ref, src_v, dst_v, rows_v, acc_sh, sems, nchunks):
    for b in range(2):
        pltpu.async_copy(g_ref.at[src_v.at[b]], rows_v.at[b], sems.at[b])

    @pl.loop(0, nchunks, step=2)
    def _(j):
        for b in range(2):
            k = j + b
            pltpu.make_async_copy(g_ref.at[src_v.at[k]], rows_v.at[b],
                                  sems.at[b]).wait()
            pltpu.sync_copy(rows_v.at[b], acc_sh.at[dst_v.at[k]], add=True)

            @pl.when(k + 2 < nchunks)
            def _():
                pltpu.async_copy(g_ref.at[src_v.at[k + 2]], rows_v.at[b],
                                 sems.at[b])


def _make_conv_kernel(nchunks):
    def _conv_kernel(g_hbm, src_hbm, dst_hbm, out_hbm, src_v, dst_v, rows_v,
                     acc_sh, sems):
        c = lax.axis_index("c")
        s = lax.axis_index("s")
        wid = s * NC + c
        pltpu.sync_copy(src_hbm.at[wid], src_v)
        pltpu.sync_copy(dst_hbm.at[wid], dst_v)
        _slab_copy(s, lambda d: g_hbm.at[d], lambda d: acc_sh.at[d])
        plsc.subcore_barrier()
        _gather_scatter_loop(g_hbm, src_v, dst_v, rows_v, acc_sh, sems,
                             nchunks)
        plsc.subcore_barrier()
        _slab_copy(s, lambda d: acc_sh.at[d], lambda d: out_hbm.at[c, d])

    return _conv_kernel


def _aggregate(g, src3, dst3):
    width = g.shape[1]
    nchunks, chunk = src3.shape[1], src3.shape[2]
    k = pl.kernel(
        _make_conv_kernel(nchunks),
        out_type=jax.ShapeDtypeStruct((NC, N_NODES, width), jnp.bfloat16),
        mesh=_vector_mesh(),
        compiler_params=_SC_PARAMS,
        scratch_types=[
            pltpu.VMEM((nchunks, chunk), jnp.int32),
            pltpu.VMEM((nchunks, chunk), jnp.int32),
            pltpu.VMEM((2, chunk, width), jnp.bfloat16),
            pltpu.VMEM_SHARED((N_NODES, width), jnp.bfloat16),
            pltpu.SemaphoreType.DMA((2,)),
        ],
    )
    return k(g, src3, dst3)


_BLK = 1000


def _dinv_block(deg_ref):
    deg = deg_ref[0, :, 0:1] + deg_ref[1, :, 0:1] + 1.0
    return lax.rsqrt(deg)


def _mm_kernel(x_ref, w_ref, o_ref):
    o_ref[...] = jnp.dot(x_ref[...], w_ref[...],
                         preferred_element_type=jnp.float32)


def _scale_kernel(h_ref, deg_ref, o_ref):
    o_ref[...] = (_dinv_block(deg_ref) * h_ref[...]).astype(jnp.bfloat16)


def _combine(acc_ref, g_ref):
    a0 = acc_ref[0].astype(jnp.float32)
    a1 = acc_ref[1].astype(jnp.float32)
    return a0 + a1 - g_ref[...].astype(jnp.float32)


def _mid_kernel(acc_ref, g_ref, deg_ref, b_ref, w_ref, o_ref):
    dinv = _dinv_block(deg_ref)
    z = dinv * _combine(acc_ref, g_ref) + b_ref[...]
    z = jnp.maximum(z, 0.0)
    o_ref[...] = (dinv * jnp.dot(z, w_ref[...],
                                 preferred_element_type=jnp.float32)
                  ).astype(jnp.bfloat16)


def _final_kernel(acc_ref, g_ref, deg_ref, b_ref, o_ref):
    dinv = _dinv_block(deg_ref)
    y = dinv * _combine(acc_ref, g_ref) + b_ref[...]
    m = jnp.max(y, axis=1, keepdims=True)
    e = jnp.exp(y - m)
    o_ref[...] = (y - m) - jnp.log(jnp.sum(e, axis=1, keepdims=True))


def _matmul1(x, w):
    din, dout = w.shape
    n = x.shape[0]
    return pl.pallas_call(
        _mm_kernel,
        grid=(n // _BLK,),
        in_specs=[
            pl.BlockSpec((_BLK, din), lambda i: (i, 0)),
            pl.BlockSpec((din, dout), lambda i: (0, 0)),
        ],
        out_specs=pl.BlockSpec((_BLK, dout), lambda i: (i, 0)),
        out_shape=jax.ShapeDtypeStruct((n, dout), jnp.float32),
    )(x, w)


def _scale_stage(h, degpair):
    n, dout = h.shape
    return pl.pallas_call(
        _scale_kernel,
        grid=(n // _BLK,),
        in_specs=[
            pl.BlockSpec((_BLK, dout), lambda i: (i, 0)),
            pl.BlockSpec((NC, _BLK, HIST_W), lambda i: (0, i, 0)),
        ],
        out_specs=pl.BlockSpec((_BLK, dout), lambda i: (i, 0)),
        out_shape=jax.ShapeDtypeStruct((n, dout), jnp.bfloat16),
    )(h, degpair)


def _mid_stage(accpair, g, degpair, b, w):
    din, dout = w.shape
    n = accpair.shape[1]
    return pl.pallas_call(
        _mid_kernel,
        grid=(n // _BLK,),
        in_specs=[
            pl.BlockSpec((NC, _BLK, din), lambda i: (0, i, 0)),
            pl.BlockSpec((_BLK, din), lambda i: (i, 0)),
            pl.BlockSpec((NC, _BLK, HIST_W), lambda i: (0, i, 0)),
            pl.BlockSpec((1, din), lambda i: (0, 0)),
            pl.BlockSpec((din, dout), lambda i: (0, 0)),
        ],
        out_specs=pl.BlockSpec((_BLK, dout), lambda i: (i, 0)),
        out_shape=jax.ShapeDtypeStruct((n, dout), jnp.bfloat16),
    )(accpair, g, degpair, b.reshape(1, din), w)


def _final_stage(accpair, g, degpair, b):
    n, dout = g.shape
    return pl.pallas_call(
        _final_kernel,
        grid=(n // _BLK,),
        in_specs=[
            pl.BlockSpec((NC, _BLK, dout), lambda i: (0, i, 0)),
            pl.BlockSpec((_BLK, dout), lambda i: (i, 0)),
            pl.BlockSpec((NC, _BLK, HIST_W), lambda i: (0, i, 0)),
            pl.BlockSpec((1, dout), lambda i: (0, 0)),
        ],
        out_specs=pl.BlockSpec((_BLK, dout), lambda i: (i, 0)),
        out_shape=jax.ShapeDtypeStruct((n, dout), jnp.float32),
    )(accpair, g, degpair, b.reshape(1, dout))


@jax.jit
def kernel(x, edge_index, W1, b1, W2, b2):
    ei = edge_index.astype(jnp.int32)
    src3 = ei[0].reshape(NW, NCHUNKS2_W, CHUNK2)
    dst3 = ei[1].reshape(NW, NCHUNKS2_W, CHUNK2)
    h = _matmul1(x, W1)
    degpair = _degree_histogram(dst3)
    g1 = _scale_stage(h, degpair)
    acc1 = _aggregate(g1, src3, dst3)
    g2 = _mid_stage(acc1, g1, degpair, b1, W2)
    acc2 = _aggregate(g2, src3, dst3)
    return _final_stage(acc2, g2, degpair, b2)


# speedup vs baseline: 37.7158x; 1.0244x over previous
import jax
import jax.numpy as jnp
from jax import lax
from jax.experimental import pallas as pl
from jax.experimental.pallas import tpu as pltpu
from jax.experimental.pallas import tpu_sc as plsc

N_NODES = 10000
N_EDGES = 320000
NC = 2
NS = 16
NW = NC * NS
CHUNK2 = 125
NCHUNKS2_W = N_EDGES // NW // CHUNK2

SLAB = 624
TAIL0 = NS * SLAB
TAIL = N_NODES - TAIL0
HIST_W = 16
HALF = 64


def _vector_mesh():
    return plsc.VectorSubcoreMesh(core_axis_name="c", subcore_axis_name="s")


_SC_PARAMS = pltpu.CompilerParams(use_tc_tiling_on_sc=False)


def _slab_copy(s, src_at, dst_at):
    row0 = pl.multiple_of(s * SLAB, 8)
    pltpu.sync_copy(src_at(pl.ds(row0, SLAB)), dst_at(pl.ds(row0, SLAB)))

    @pl.when(s == 0)
    def _():
        pltpu.sync_copy(src_at(pl.ds(TAIL0, TAIL)), dst_at(pl.ds(TAIL0, TAIL)))


def _hist_kernel(dst_hbm, ones_hbm, zeros_hbm, out_hbm, dst_v, ones_v, deg_sh):
    c = lax.axis_index("c")
    s = lax.axis_index("s")
    wid = s * NC + c
    pltpu.sync_copy(dst_hbm.at[wid], dst_v)
    pltpu.sync_copy(ones_hbm, ones_v)
    _slab_copy(s, lambda d: zeros_hbm.at[d], lambda d: deg_sh.at[d])
    plsc.subcore_barrier()

    @pl.loop(0, NCHUNKS2_W)
    def _(j):
        pltpu.sync_copy(ones_v, deg_sh.at[dst_v.at[j]], add=True)

    plsc.subcore_barrier()
    _slab_copy(s, lambda d: deg_sh.at[d], lambda d: out_hbm.at[c, d])


def _degree_histogram(dst3w):
    ones = jnp.ones((CHUNK2, HIST_W), jnp.float32)
    zeros = jnp.zeros((N_NODES, HIST_W), jnp.float32)
    k = pl.kernel(
        _hist_kernel,
        out_type=jax.ShapeDtypeStruct((NC, N_NODES, HIST_W), jnp.float32),
        mesh=_vector_mesh(),
        compiler_params=_SC_PARAMS,
        scratch_types=[
            pltpu.VMEM((NCHUNKS2_W, CHUNK2), jnp.int32),
            pltpu.VMEM((CHUNK2, HIST_W), jnp.float32),
            pltpu.VMEM_SHARED((N_NODES, HIST_W), jnp.float32),
        ],
    )
    return k(dst3w, ones, zeros)


def _gather_scatter_loop(g_ref, src_v, dst_v, rows_v, acc_sh, sems, nchunks):
    for b in range(2):
        pltpu.async_copy(g_ref.at[src_v.at[b]], rows_v.at[b], sems.at[b])

    @pl.loop(0, nchunks, step=2)
    def _(j):
        for b in range(2):
            k = j + b
            pltpu.make_async_copy(g_ref.at[src_v.at[k]], rows_v.at[b],
                                  sems.at[b]).wait()
            pltpu.sync_copy(rows_v.at[b], acc_sh.at[dst_v.at[k]], add=True)

            @pl.when(k + 2 < nchunks)
            def _():
                pltpu.async_copy(g_ref.at[src_v.at[k + 2]], rows_v.at[b],
                                 sems.at[b])


def _make_conv_kernel(nchunks):
    def _conv_kernel(g_hbm, src_hbm, dst_hbm, out_hbm, src_v, dst_v, rows_v,
                     acc_sh, sems):
        c = lax.axis_index("c")
        s = lax.axis_index("s")
        wid = s * NC + c
        pltpu.sync_copy(src_hbm.at[wid], src_v)
        pltpu.sync_copy(dst_hbm.at[wid], dst_v)
        _slab_copy(s, lambda d: g_hbm.at[d], lambda d: acc_sh.at[d])
        plsc.subcore_barrier()
        _gather_scatter_loop(g_hbm, src_v, dst_v, rows_v, acc_sh, sems,
                             nchunks)
        plsc.subcore_barrier()
        _slab_copy(s, lambda d: acc_sh.at[d], lambda d: out_hbm.at[c, d])

    return _conv_kernel


def _aggregate(g, src3, dst3):
    width = g.shape[1]
    nchunks, chunk = src3.shape[1], src3.shape[2]
    k = pl.kernel(
        _make_conv_kernel(nchunks),
        out_type=jax.ShapeDtypeStruct((NC, N_NODES, width), jnp.bfloat16),
        mesh=_vector_mesh(),
        compiler_params=_SC_PARAMS,
        scratch_types=[
            pltpu.VMEM((nchunks, chunk), jnp.int32),
            pltpu.VMEM((nchunks, chunk), jnp.int32),
            pltpu.VMEM((2, chunk, width), jnp.bfloat16),
            pltpu.VMEM_SHARED((N_NODES, width), jnp.bfloat16),
            pltpu.SemaphoreType.DMA((2,)),
        ],
    )
    return k(g, src3, dst3)


_BLK = 2000


def _dinv_block(deg_ref):
    deg = deg_ref[0, :, 0:1] + deg_ref[1, :, 0:1] + 1.0
    return lax.rsqrt(deg)


def _mm_kernel(x_ref, w_ref, o_ref):
    o_ref[...] = jnp.dot(x_ref[...], w_ref[...],
                         preferred_element_type=jnp.float32)


def _scale_kernel(h_ref, deg_ref, o_ref):
    o_ref[...] = (_dinv_block(deg_ref) * h_ref[...]).astype(jnp.bfloat16)


def _combine(acc_ref, g_ref):
    a0 = acc_ref[0].astype(jnp.float32)
    a1 = acc_ref[1].astype(jnp.float32)
    return a0 + a1 - g_ref[...].astype(jnp.float32)


def _mid_kernel(acc_ref, g_ref, deg_ref, b_ref, w_ref, o_ref):
    dinv = _dinv_block(deg_ref)
    z = dinv * _combine(acc_ref, g_ref) + b_ref[...]
    z = jnp.maximum(z, 0.0)
    o_ref[...] = (dinv * jnp.dot(z, w_ref[...],
                                 preferred_element_type=jnp.float32)
                  ).astype(jnp.bfloat16)


def _final_kernel(acc_ref, g_ref, deg_ref, b_ref, o_ref):
    dinv = _dinv_block(deg_ref)
    y = dinv * _combine(acc_ref, g_ref) + b_ref[...]
    m = jnp.max(y, axis=1, keepdims=True)
    e = jnp.exp(y - m)
    o_ref[...] = (y - m) - jnp.log(jnp.sum(e, axis=1, keepdims=True))


def _matmul1(x, w):
    din, dout = w.shape
    n = x.shape[0]
    return pl.pallas_call(
        _mm_kernel,
        grid=(n // _BLK,),
        in_specs=[
            pl.BlockSpec((_BLK, din), lambda i: (i, 0)),
            pl.BlockSpec((din, dout), lambda i: (0, 0)),
        ],
        out_specs=pl.BlockSpec((_BLK, dout), lambda i: (i, 0)),
        out_shape=jax.ShapeDtypeStruct((n, dout), jnp.float32),
    )(x, w)


def _scale_stage(h, degpair):
    n, dout = h.shape
    return pl.pallas_call(
        _scale_kernel,
        grid=(n // _BLK,),
        in_specs=[
            pl.BlockSpec((_BLK, dout), lambda i: (i, 0)),
            pl.BlockSpec((NC, _BLK, HIST_W), lambda i: (0, i, 0)),
        ],
        out_specs=pl.BlockSpec((_BLK, dout), lambda i: (i, 0)),
        out_shape=jax.ShapeDtypeStruct((n, dout), jnp.bfloat16),
    )(h, degpair)


def _mid_stage(accpair, g, degpair, b, w):
    din, dout = w.shape
    n = accpair.shape[1]
    return pl.pallas_call(
        _mid_kernel,
        grid=(n // _BLK,),
        in_specs=[
            pl.BlockSpec((NC, _BLK, din), lambda i: (0, i, 0)),
            pl.BlockSpec((_BLK, din), lambda i: (i, 0)),
            pl.BlockSpec((NC, _BLK, HIST_W), lambda i: (0, i, 0)),
            pl.BlockSpec((1, din), lambda i: (0, 0)),
            pl.BlockSpec((din, dout), lambda i: (0, 0)),
        ],
        out_specs=pl.BlockSpec((_BLK, dout), lambda i: (i, 0)),
        out_shape=jax.ShapeDtypeStruct((n, dout), jnp.bfloat16),
    )(accpair, g, degpair, b.reshape(1, din), w)


def _final_stage(accpair, g, degpair, b):
    n, dout = g.shape
    return pl.pallas_call(
        _final_kernel,
        grid=(n // _BLK,),
        in_specs=[
            pl.BlockSpec((NC, _BLK, dout), lambda i: (0, i, 0)),
            pl.BlockSpec((_BLK, dout), lambda i: (i, 0)),
            pl.BlockSpec((NC, _BLK, HIST_W), lambda i: (0, i, 0)),
            pl.BlockSpec((1, dout), lambda i: (0, 0)),
        ],
        out_specs=pl.BlockSpec((_BLK, dout), lambda i: (i, 0)),
        out_shape=jax.ShapeDtypeStruct((n, dout), jnp.float32),
    )(accpair, g, degpair, b.reshape(1, dout))


@jax.jit
def kernel(x, edge_index, W1, b1, W2, b2):
    ei = edge_index.astype(jnp.int32)
    src3 = ei[0].reshape(NW, NCHUNKS2_W, CHUNK2)
    dst3 = ei[1].reshape(NW, NCHUNKS2_W, CHUNK2)
    h = _matmul1(x, W1)
    degpair = _degree_histogram(dst3)
    g1 = _scale_stage(h, degpair)
    acc1 = _aggregate(g1, src3, dst3)
    g2 = _mid_stage(acc1, g1, degpair, b1, W2)
    acc2 = _aggregate(g2, src3, dst3)
    return _final_stage(acc2, g2, degpair, b2)


# speedup vs baseline: 38.0424x; 1.0087x over previous
import jax
import jax.numpy as jnp
from jax import lax
from jax.experimental import pallas as pl
from jax.experimental.pallas import tpu as pltpu
from jax.experimental.pallas import tpu_sc as plsc

N_NODES = 10000
N_EDGES = 320000
NC = 2
NS = 16
NW = NC * NS
CHUNK2 = 125
NCHUNKS2_W = N_EDGES // NW // CHUNK2

SLAB = 624
TAIL0 = NS * SLAB
TAIL = N_NODES - TAIL0
HIST_W = 16
HALF = 64


def _vector_mesh():
    return plsc.VectorSubcoreMesh(core_axis_name="c", subcore_axis_name="s")


_SC_PARAMS = pltpu.CompilerParams(use_tc_tiling_on_sc=False)


def _slab_copy(s, src_at, dst_at):
    row0 = pl.multiple_of(s * SLAB, 8)
    pltpu.sync_copy(src_at(pl.ds(row0, SLAB)), dst_at(pl.ds(row0, SLAB)))

    @pl.when(s == 0)
    def _():
        pltpu.sync_copy(src_at(pl.ds(TAIL0, TAIL)), dst_at(pl.ds(TAIL0, TAIL)))


def _hist_kernel(dst_hbm, ones_hbm, zeros_hbm, out_hbm, dst_v, ones_v, deg_sh):
    c = lax.axis_index("c")
    s = lax.axis_index("s")
    wid = s * NC + c
    pltpu.sync_copy(dst_hbm.at[wid], dst_v)
    pltpu.sync_copy(ones_hbm, ones_v)
    _slab_copy(s, lambda d: zeros_hbm.at[d], lambda d: deg_sh.at[d])
    plsc.subcore_barrier()

    @pl.loop(0, NCHUNKS2_W)
    def _(j):
        pltpu.sync_copy(ones_v, deg_sh.at[dst_v.at[j]], add=True)

    plsc.subcore_barrier()
    _slab_copy(s, lambda d: deg_sh.at[d], lambda d: out_hbm.at[c, d])


def _degree_histogram(dst3w):
    ones = jnp.ones((CHUNK2, HIST_W), jnp.float32)
    zeros = jnp.zeros((N_NODES, HIST_W), jnp.float32)
    k = pl.kernel(
        _hist_kernel,
        out_type=jax.ShapeDtypeStruct((NC, N_NODES, HIST_W), jnp.float32),
        mesh=_vector_mesh(),
        compiler_params=_SC_PARAMS,
        scratch_types=[
            pltpu.VMEM((NCHUNKS2_W, CHUNK2), jnp.int32),
            pltpu.VMEM((CHUNK2, HIST_W), jnp.float32),
            pltpu.VMEM_SHARED((N_NODES, HIST_W), jnp.float32),
        ],
    )
    return k(dst3w, ones, zeros)


def _gather_scatter_loop(g_ref, src_v, dst_v, rows_v, acc_sh, sems, nchunks):
    for b in range(2):
        pltpu.async_copy(g_ref.at[src_v.at[b]], rows_v.at[b], sems.at[b])

    @pl.loop(0, nchunks, step=2)
    def _(j):
        for b in range(2):
            k = j + b
            pltpu.make_async_copy(g_ref.at[src_v.at[k]], rows_v.at[b],
                                  sems.at[b]).wait()
            pltpu.sync_copy(rows_v.at[b], acc_sh.at[dst_v.at[k]], add=True)

            @pl.when(k + 2 < nchunks)
            def _():
                pltpu.async_copy(g_ref.at[src_v.at[k + 2]], rows_v.at[b],
                                 sems.at[b])


def _make_conv_kernel(nchunks):
    def _conv_kernel(g_hbm, src_hbm, dst_hbm, out_hbm, src_v, dst_v, rows_v,
                     acc_sh, sems):
        c = lax.axis_index("c")
        s = lax.axis_index("s")
        wid = s * NC + c
        pltpu.sync_copy(src_hbm.at[wid], src_v)
        pltpu.sync_copy(dst_hbm.at[wid], dst_v)
        _slab_copy(s, lambda d: g_hbm.at[d], lambda d: acc_sh.at[d])
        plsc.subcore_barrier()
        _gather_scatter_loop(g_hbm, src_v, dst_v, rows_v, acc_sh, sems,
                             nchunks)
        plsc.subcore_barrier()
        _slab_copy(s, lambda d: acc_sh.at[d], lambda d: out_hbm.at[c, d])

    return _conv_kernel


def _aggregate(g, src3, dst3):
    width = g.shape[1]
    nchunks, chunk = src3.shape[1], src3.shape[2]
    k = pl.kernel(
        _make_conv_kernel(nchunks),
        out_type=jax.ShapeDtypeStruct((NC, N_NODES, width), jnp.bfloat16),
        mesh=_vector_mesh(),
        compiler_params=_SC_PARAMS,
        scratch_types=[
            pltpu.VMEM((nchunks, chunk), jnp.int32),
            pltpu.VMEM((nchunks, chunk), jnp.int32),
            pltpu.VMEM((2, chunk, width), jnp.bfloat16),
            pltpu.VMEM_SHARED((N_NODES, width), jnp.bfloat16),
            pltpu.SemaphoreType.DMA((2,)),
        ],
    )
    return k(g, src3, dst3)


_BLK = 5000


def _dinv_block(deg_ref):
    deg = deg_ref[0, :, 0:1] + deg_ref[1, :, 0:1] + 1.0
    return lax.rsqrt(deg)


def _mm_kernel(x_ref, w_ref, o_ref):
    o_ref[...] = jnp.dot(x_ref[...], w_ref[...],
                         preferred_element_type=jnp.float32)


def _scale_kernel(h_ref, deg_ref, o_ref):
    o_ref[...] = (_dinv_block(deg_ref) * h_ref[...]).astype(jnp.bfloat16)


def _combine(acc_ref, g_ref):
    a0 = acc_ref[0].astype(jnp.float32)
    a1 = acc_ref[1].astype(jnp.float32)
    return a0 + a1 - g_ref[...].astype(jnp.float32)


def _mid_kernel(acc_ref, g_ref, deg_ref, b_ref, w_ref, o_ref):
    dinv = _dinv_block(deg_ref)
    z = dinv * _combine(acc_ref, g_ref) + b_ref[...]
    z = jnp.maximum(z, 0.0)
    o_ref[...] = (dinv * jnp.dot(z, w_ref[...],
                                 preferred_element_type=jnp.float32)
                  ).astype(jnp.bfloat16)


def _final_kernel(acc_ref, g_ref, deg_ref, b_ref, o_ref):
    dinv = _dinv_block(deg_ref)
    y = dinv * _combine(acc_ref, g_ref) + b_ref[...]
    m = jnp.max(y, axis=1, keepdims=True)
    e = jnp.exp(y - m)
    o_ref[...] = (y - m) - jnp.log(jnp.sum(e, axis=1, keepdims=True))


def _matmul1(x, w):
    din, dout = w.shape
    n = x.shape[0]
    return pl.pallas_call(
        _mm_kernel,
        grid=(n // _BLK,),
        in_specs=[
            pl.BlockSpec((_BLK, din), lambda i: (i, 0)),
            pl.BlockSpec((din, dout), lambda i: (0, 0)),
        ],
        out_specs=pl.BlockSpec((_BLK, dout), lambda i: (i, 0)),
        out_shape=jax.ShapeDtypeStruct((n, dout), jnp.float32),
    )(x, w)


def _scale_stage(h, degpair):
    n, dout = h.shape
    return pl.pallas_call(
        _scale_kernel,
        grid=(n // _BLK,),
        in_specs=[
            pl.BlockSpec((_BLK, dout), lambda i: (i, 0)),
            pl.BlockSpec((NC, _BLK, HIST_W), lambda i: (0, i, 0)),
        ],
        out_specs=pl.BlockSpec((_BLK, dout), lambda i: (i, 0)),
        out_shape=jax.ShapeDtypeStruct((n, dout), jnp.bfloat16),
    )(h, degpair)


def _mid_stage(accpair, g, degpair, b, w):
    din, dout = w.shape
    n = accpair.shape[1]
    return pl.pallas_call(
        _mid_kernel,
        grid=(n // _BLK,),
        in_specs=[
            pl.BlockSpec((NC, _BLK, din), lambda i: (0, i, 0)),
            pl.BlockSpec((_BLK, din), lambda i: (i, 0)),
            pl.BlockSpec((NC, _BLK, HIST_W), lambda i: (0, i, 0)),
            pl.BlockSpec((1, din), lambda i: (0, 0)),
            pl.BlockSpec((din, dout), lambda i: (0, 0)),
        ],
        out_specs=pl.BlockSpec((_BLK, dout), lambda i: (i, 0)),
        out_shape=jax.ShapeDtypeStruct((n, dout), jnp.bfloat16),
    )(accpair, g, degpair, b.reshape(1, din), w)


def _final_stage(accpair, g, degpair, b):
    n, dout = g.shape
    return pl.pallas_call(
        _final_kernel,
        grid=(n // _BLK,),
        in_specs=[
            pl.BlockSpec((NC, _BLK, dout), lambda i: (0, i, 0)),
            pl.BlockSpec((_BLK, dout), lambda i: (i, 0)),
            pl.BlockSpec((NC, _BLK, HIST_W), lambda i: (0, i, 0)),
            pl.BlockSpec((1, dout), lambda i: (0, 0)),
        ],
        out_specs=pl.BlockSpec((_BLK, dout), lambda i: (i, 0)),
        out_shape=jax.ShapeDtypeStruct((n, dout), jnp.float32),
    )(accpair, g, degpair, b.reshape(1, dout))


@jax.jit
def kernel(x, edge_index, W1, b1, W2, b2):
    ei = edge_index.astype(jnp.int32)
    src3 = ei[0].reshape(NW, NCHUNKS2_W, CHUNK2)
    dst3 = ei[1].reshape(NW, NCHUNKS2_W, CHUNK2)
    h = _matmul1(x, W1)
    degpair = _degree_histogram(dst3)
    g1 = _scale_stage(h, degpair)
    acc1 = _aggregate(g1, src3, dst3)
    g2 = _mid_stage(acc1, g1, degpair, b1, W2)
    acc2 = _aggregate(g2, src3, dst3)
    return _final_stage(acc2, g2, degpair, b2)
